# Initial kernel scaffold; baseline (speedup 1.0000x reference)
import jax
import jax.numpy as jnp
from jax.experimental import pallas as pl


def kernel(syntax, emb_table, pos_emb):
    raise NotImplementedError("write your pallas kernel here")


# speedup vs baseline: 26.1776x; 26.1776x over previous
import functools

import jax
import jax.numpy as jnp
from jax import lax
from jax.experimental import pallas as pl
from jax.experimental.pallas import tpu as pltpu
from jax.experimental.pallas import tpu_sc as plsc

_B, _L, _D, _E = 4096, 50, 20, 32
_N = _B * _L
_NW = 32
_RPW = _N // _NW
_C = 64
_CHUNKS = _RPW // _C
_IPC = _C * _D
_GSZ = 128
_NG = _IPC // _GSZ


def _sc_body(syntax_hbm, table_hbm, pos_hbm, out_hbm,
             idx_v, rows_v, out_v, pos_v, sem):
    wid = lax.axis_index("s") * 2 + lax.axis_index("c")
    pltpu.sync_copy(pos_hbm, pos_v)

    def chunk_body(g, _):
        ibase = (wid * _CHUNKS + g) * _IPC
        pltpu.sync_copy(syntax_hbm.at[pl.ds(ibase, _IPC)], idx_v)
        copies = [
            pltpu.async_copy(
                table_hbm.at[idx_v.at[pl.ds(j * _GSZ, _GSZ)]],
                rows_v.at[pl.ds(j * _GSZ, _GSZ)],
                sem,
            )
            for j in range(_NG)
        ]
        for cp in copies:
            cp.wait()

        def row_body(c, _):
            base = c * _D
            acc0 = jnp.zeros((16,), jnp.float32)
            acc1 = jnp.zeros((16,), jnp.float32)
            for d in range(_D):
                acc0 += rows_v[base + d, pl.ds(0, 16)] * pos_v[d, pl.ds(0, 16)]
                acc1 += rows_v[base + d, pl.ds(16, 16)] * pos_v[d, pl.ds(16, 16)]
            out_v[c, pl.ds(0, 16)] = acc0
            out_v[c, pl.ds(16, 16)] = acc1
            return 0

        lax.fori_loop(0, _C, row_body, 0, unroll=False)
        pltpu.sync_copy(out_v, out_hbm.at[pl.ds(wid * _RPW + g * _C, _C)])
        return 0

    lax.fori_loop(0, _CHUNKS, chunk_body, 0, unroll=False)


@jax.jit
def _syntax_embed(syntax_flat, emb_table, pos_emb):
    mesh = plsc.VectorSubcoreMesh(core_axis_name="c", subcore_axis_name="s")
    return pl.kernel(
        _sc_body,
        out_type=jax.ShapeDtypeStruct((_N, _E), jnp.float32),
        mesh=mesh,
        compiler_params=pltpu.CompilerParams(use_tc_tiling_on_sc=False),
        scratch_types=[
            pltpu.VMEM((_IPC,), jnp.int32),
            pltpu.VMEM((_IPC, _E), jnp.float32),
            pltpu.VMEM((_C, _E), jnp.float32),
            pltpu.VMEM((_D, _E), jnp.float32),
            pltpu.SemaphoreType.DMA,
        ],
    )(syntax_flat, emb_table, pos_emb)


def kernel(syntax, emb_table, pos_emb):
    syntax_flat = syntax.reshape(_N * _D)
    out = _syntax_embed(syntax_flat, emb_table, pos_emb)
    return out.reshape(_B, _L, _E)


# speedup vs baseline: 37.0145x; 1.4140x over previous
import functools

import jax
import jax.numpy as jnp
from jax import lax
from jax.experimental import pallas as pl
from jax.experimental.pallas import tpu as pltpu
from jax.experimental.pallas import tpu_sc as plsc

_B, _L, _D, _E = 4096, 50, 20, 32
_N = _B * _L
_NW = 32
_RPW = _N // _NW
_C = 64
_CHUNKS = _RPW // _C
_IPC = _C * _D
_GSZ = 128
_NG = _IPC // _GSZ
_G = 4


def _sc_body(syntax_hbm, table_hbm, pos_hbm, out_hbm,
             idx_a, idx_b, rows_a, rows_b, out_a, out_b, pos_v,
             gsem_a, gsem_b, osem_a, osem_b):
    wid = lax.axis_index("s") * 2 + lax.axis_index("c")
    ibase0 = wid * _CHUNKS * _IPC
    obase0 = wid * _RPW
    pltpu.sync_copy(pos_hbm, pos_v)

    def fire(g, idx_v, rows_v, gsem):
        pltpu.sync_copy(syntax_hbm.at[pl.ds(ibase0 + g * _IPC, _IPC)], idx_v)
        for j in range(_NG):
            pltpu.async_copy(
                table_hbm.at[idx_v.at[pl.ds(j * _GSZ, _GSZ)]],
                rows_v.at[pl.ds(j * _GSZ, _GSZ)],
                gsem,
            )

    def drain(idx_v, rows_v, gsem):
        for j in range(_NG):
            pltpu.make_async_copy(
                table_hbm.at[idx_v.at[pl.ds(j * _GSZ, _GSZ)]],
                rows_v.at[pl.ds(j * _GSZ, _GSZ)],
                gsem,
            ).wait()

    def compute(rows_v, out_v):
        def group_body(t, _):
            base = t * (_G * _D)
            acc = [[jnp.zeros((16,), jnp.float32) for _ in range(2)]
                   for _ in range(_G)]
            for d in range(_D):
                p0 = pos_v[d, pl.ds(0, 16)]
                p1 = pos_v[d, pl.ds(16, 16)]
                for r in range(_G):
                    row = base + r * _D + d
                    acc[r][0] += rows_v[row, pl.ds(0, 16)] * p0
                    acc[r][1] += rows_v[row, pl.ds(16, 16)] * p1
            for r in range(_G):
                out_v[t * _G + r, pl.ds(0, 16)] = acc[r][0]
                out_v[t * _G + r, pl.ds(16, 16)] = acc[r][1]
            return 0

        lax.fori_loop(0, _C // _G, group_body, 0, unroll=False)

    def out_wait(out_v, osem):
        pltpu.make_async_copy(out_v, out_hbm.at[pl.ds(obase0, _C)], osem).wait()

    def out_send(g, out_v, osem):
        pltpu.async_copy(out_v, out_hbm.at[pl.ds(obase0 + g * _C, _C)], osem)

    fire(0, idx_a, rows_a, gsem_a)

    def pair_body(k, _):
        g0 = 2 * k
        fire(g0 + 1, idx_b, rows_b, gsem_b)
        drain(idx_a, rows_a, gsem_a)

        @pl.when(k >= 1)
        def _():
            out_wait(out_a, osem_a)

        compute(rows_a, out_a)
        out_send(g0, out_a, osem_a)

        fire(g0 + 2, idx_a, rows_a, gsem_a)
        drain(idx_b, rows_b, gsem_b)

        @pl.when(k >= 1)
        def _():
            out_wait(out_b, osem_b)

        compute(rows_b, out_b)
        out_send(g0 + 1, out_b, osem_b)
        return 0

    lax.fori_loop(0, _CHUNKS // 2 - 1, pair_body, 0, unroll=False)

    fire(_CHUNKS - 1, idx_b, rows_b, gsem_b)
    drain(idx_a, rows_a, gsem_a)
    out_wait(out_a, osem_a)
    compute(rows_a, out_a)
    out_send(_CHUNKS - 2, out_a, osem_a)
    drain(idx_b, rows_b, gsem_b)
    out_wait(out_b, osem_b)
    compute(rows_b, out_b)
    out_send(_CHUNKS - 1, out_b, osem_b)
    out_wait(out_a, osem_a)
    out_wait(out_b, osem_b)


@jax.jit
def _syntax_embed(syntax_flat, emb_table, pos_emb):
    mesh = plsc.VectorSubcoreMesh(core_axis_name="c", subcore_axis_name="s")
    return pl.kernel(
        _sc_body,
        out_type=jax.ShapeDtypeStruct((_N, _E), jnp.float32),
        mesh=mesh,
        compiler_params=pltpu.CompilerParams(use_tc_tiling_on_sc=False),
        scratch_types=[
            pltpu.VMEM((_IPC,), jnp.int32),
            pltpu.VMEM((_IPC,), jnp.int32),
            pltpu.VMEM((_IPC, _E), jnp.float32),
            pltpu.VMEM((_IPC, _E), jnp.float32),
            pltpu.VMEM((_C, _E), jnp.float32),
            pltpu.VMEM((_C, _E), jnp.float32),
            pltpu.VMEM((_D, _E), jnp.float32),
            pltpu.SemaphoreType.DMA,
            pltpu.SemaphoreType.DMA,
            pltpu.SemaphoreType.DMA,
            pltpu.SemaphoreType.DMA,
        ],
    )(syntax_flat, emb_table, pos_emb)


def kernel(syntax, emb_table, pos_emb):
    syntax_flat = syntax.reshape(_N * _D)
    out = _syntax_embed(syntax_flat, emb_table, pos_emb)
    return out.reshape(_B, _L, _E)


# speedup vs baseline: 46.9894x; 1.2695x over previous
import functools

import jax
import jax.numpy as jnp
from jax import lax
from jax.experimental import pallas as pl
from jax.experimental.pallas import tpu as pltpu
from jax.experimental.pallas import tpu_sc as plsc

_B, _L, _D, _E = 4096, 50, 20, 32
_NW = 32
_BPW = _B // _NW
_HB = _BPW // 2
_IPH = _HB * _D
_GSZ = 128
_NG = _IPH // _GSZ
_G = 4


def _sc_body(syntax_hbm, table_hbm, pos_hbm, out_hbm,
             idx_a, idx_b, slab_a, slab_b, acc_a, acc_b, pos_v,
             gsem_a, gsem_b, osem_a, osem_b):
    wid = lax.axis_index("s") * 2 + lax.axis_index("c")
    b0 = wid * _BPW
    pltpu.sync_copy(pos_hbm, pos_v)

    def fire(l, hb, idx_v, slab, gsem):
        base = (l * _B + b0 + hb * _HB) * _D
        pltpu.sync_copy(syntax_hbm.at[pl.ds(base, _IPH)], idx_v)
        for j in range(_NG):
            pltpu.async_copy(
                table_hbm.at[idx_v.at[pl.ds(j * _GSZ, _GSZ)]],
                slab.at[pl.ds(j * _GSZ, _GSZ)],
                gsem,
            )

    def drain(idx_v, slab, gsem):
        for j in range(_NG):
            pltpu.make_async_copy(
                table_hbm.at[idx_v.at[pl.ds(j * _GSZ, _GSZ)]],
                slab.at[pl.ds(j * _GSZ, _GSZ)],
                gsem,
            ).wait()

    e_lo = lax.iota(jnp.int32, 16)
    e_hi = e_lo + 16

    def compute(slab, acc, col0):
        def group_body(g, _):
            base = g * (_G * _D)
            racc = [[jnp.zeros((16,), jnp.float32) for _ in range(2)]
                    for _ in range(_G)]
            for d in range(_D):
                p0 = pos_v[d, pl.ds(0, 16)]
                p1 = pos_v[d, pl.ds(16, 16)]
                for r in range(_G):
                    row = base + r * _D + d
                    racc[r][0] += slab[row, pl.ds(0, 16)] * p0
                    racc[r][1] += slab[row, pl.ds(16, 16)] * p1
            for r in range(_G):
                j = jnp.full((16,), col0 + g * _G + r, jnp.int32)
                plsc.store_scatter(acc, [e_lo, j], racc[r][0])
                plsc.store_scatter(acc, [e_hi, j], racc[r][1])
            return 0

        lax.fori_loop(0, _HB // _G, group_body, 0, unroll=False)

    def out_wait(acc, osem):
        for k in range(4):
            pltpu.make_async_copy(
                acc.at[pl.ds(8 * k, 8)], out_hbm.at[0, k, wid], osem,
            ).wait()

    def out_send(l, acc, osem):
        for k in range(4):
            pltpu.async_copy(
                acc.at[pl.ds(8 * k, 8)], out_hbm.at[l, k, wid], osem,
            )

    fire(0, 0, idx_a, slab_a, gsem_a)

    def pair_body(t, _):
        l0 = 2 * t
        l1 = l0 + 1
        fire(l0, 1, idx_b, slab_b, gsem_b)
        drain(idx_a, slab_a, gsem_a)

        @pl.when(t >= 1)
        def _():
            out_wait(acc_a, osem_a)

        compute(slab_a, acc_a, 0)

        fire(l1, 0, idx_a, slab_a, gsem_a)
        drain(idx_b, slab_b, gsem_b)
        compute(slab_b, acc_a, _HB)
        out_send(l0, acc_a, osem_a)

        fire(l1, 1, idx_b, slab_b, gsem_b)
        drain(idx_a, slab_a, gsem_a)

        @pl.when(t >= 1)
        def _():
            out_wait(acc_b, osem_b)

        compute(slab_a, acc_b, 0)

        @pl.when(t < _L // 2 - 1)
        def _():
            fire(l0 + 2, 0, idx_a, slab_a, gsem_a)

        drain(idx_b, slab_b, gsem_b)
        compute(slab_b, acc_b, _HB)
        out_send(l1, acc_b, osem_b)
        return 0

    lax.fori_loop(0, _L // 2, pair_body, 0, unroll=False)
    out_wait(acc_a, osem_a)
    out_wait(acc_b, osem_b)


@jax.jit
def _syntax_embed(syntax_lmaj, emb_table, pos_emb):
    mesh = plsc.VectorSubcoreMesh(core_axis_name="c", subcore_axis_name="s")
    return pl.kernel(
        _sc_body,
        out_type=jax.ShapeDtypeStruct((_L, 4, _NW, 8, 128), jnp.float32),
        mesh=mesh,
        compiler_params=pltpu.CompilerParams(use_tc_tiling_on_sc=False, needs_layout_passes=False),
        scratch_types=[
            pltpu.VMEM((_IPH,), jnp.int32),
            pltpu.VMEM((_IPH,), jnp.int32),
            pltpu.VMEM((_IPH, _E), jnp.float32),
            pltpu.VMEM((_IPH, _E), jnp.float32),
            pltpu.VMEM((_E, _BPW), jnp.float32),
            pltpu.VMEM((_E, _BPW), jnp.float32),
            pltpu.VMEM((_D, _E), jnp.float32),
            pltpu.SemaphoreType.DMA,
            pltpu.SemaphoreType.DMA,
            pltpu.SemaphoreType.DMA,
            pltpu.SemaphoreType.DMA,
        ],
    )(syntax_lmaj, emb_table, pos_emb)


def kernel(syntax, emb_table, pos_emb):
    syntax_lmaj = syntax.transpose(1, 0, 2).reshape(_L * _B * _D)
    out5 = _syntax_embed(syntax_lmaj, emb_table, pos_emb)
    out = out5.transpose(2, 4, 0, 1, 3).reshape(_B, _L, _E)
    return out


# speedup vs baseline: 57.2745x; 1.2189x over previous
import functools

import jax
import jax.numpy as jnp
from jax import lax
from jax.experimental import pallas as pl
from jax.experimental.pallas import tpu as pltpu
from jax.experimental.pallas import tpu_sc as plsc

_B, _L, _D, _E = 4096, 50, 20, 32
_NW = 32
_BPW = _B // _NW
_HB = _BPW // 2
_IPH = _HB * _D
_GSZ = 128
_NG = _IPH // _GSZ
_G = 4


def _sc_body(syntax_hbm, table_hbm, pos_hbm, out_hbm,
             idx_a, idx_b, slab_a, slab_b, acc_a, acc_b, pos_v,
             gsem_a, gsem_b, osem_a, osem_b):
    wid = lax.axis_index("s") * 2 + lax.axis_index("c")
    b0 = wid * _BPW
    pltpu.sync_copy(pos_hbm, pos_v)

    def fire(l, hb, idx_v, slab, gsem):
        base = (l * _B + b0 + hb * _HB) * _D
        pltpu.sync_copy(syntax_hbm.at[pl.ds(base, _IPH)], idx_v)
        for j in range(_NG):
            pltpu.async_copy(
                table_hbm.at[idx_v.at[pl.ds(j * _GSZ, _GSZ)]],
                slab.at[pl.ds(j * _GSZ, _GSZ)],
                gsem,
            )

    def drain(idx_v, slab, gsem):
        for j in range(_NG):
            pltpu.make_async_copy(
                table_hbm.at[idx_v.at[pl.ds(j * _GSZ, _GSZ)]],
                slab.at[pl.ds(j * _GSZ, _GSZ)],
                gsem,
            ).wait()

    e_even = lax.iota(jnp.int32, 16) * 2
    e_odd = e_even + 1

    def compute(slab, acc, col0):
        def group_body(g, _):
            base = g * (_G * _D)
            racc = [jnp.zeros((32,), jnp.bfloat16) for _ in range(_G)]
            for d in range(_D):
                p = pos_v[d, pl.ds(0, _E)]
                for r in range(_G):
                    racc[r] += slab[base + r * _D + d, pl.ds(0, _E)] * p
            for r in range(_G):
                j = jnp.full((16,), col0 + g * _G + r, jnp.int32)
                v_even, v_odd = plsc.unpack(racc[r],
                                            format=plsc.PackFormat.INTERLEAVED)
                plsc.store_scatter(acc, [e_even, j], v_even)
                plsc.store_scatter(acc, [e_odd, j], v_odd)
            return 0

        lax.fori_loop(0, _HB // _G, group_body, 0, unroll=False)

    def out_wait(acc, osem):
        for k in range(4):
            pltpu.make_async_copy(
                acc.at[pl.ds(8 * k, 8)], out_hbm.at[0, k, wid], osem,
            ).wait()

    def out_send(l, acc, osem):
        for k in range(4):
            pltpu.async_copy(
                acc.at[pl.ds(8 * k, 8)], out_hbm.at[l, k, wid], osem,
            )

    fire(0, 0, idx_a, slab_a, gsem_a)

    def pair_body(t, _):
        l0 = 2 * t
        l1 = l0 + 1
        fire(l0, 1, idx_b, slab_b, gsem_b)
        drain(idx_a, slab_a, gsem_a)

        @pl.when(t >= 1)
        def _():
            out_wait(acc_a, osem_a)

        compute(slab_a, acc_a, 0)

        fire(l1, 0, idx_a, slab_a, gsem_a)
        drain(idx_b, slab_b, gsem_b)
        compute(slab_b, acc_a, _HB)
        out_send(l0, acc_a, osem_a)

        fire(l1, 1, idx_b, slab_b, gsem_b)
        drain(idx_a, slab_a, gsem_a)

        @pl.when(t >= 1)
        def _():
            out_wait(acc_b, osem_b)

        compute(slab_a, acc_b, 0)

        @pl.when(t < _L // 2 - 1)
        def _():
            fire(l0 + 2, 0, idx_a, slab_a, gsem_a)

        drain(idx_b, slab_b, gsem_b)
        compute(slab_b, acc_b, _HB)
        out_send(l1, acc_b, osem_b)
        return 0

    lax.fori_loop(0, _L // 2, pair_body, 0, unroll=False)
    out_wait(acc_a, osem_a)
    out_wait(acc_b, osem_b)


@jax.jit
def _syntax_embed(syntax_lmaj, emb_table, pos_emb):
    mesh = plsc.VectorSubcoreMesh(core_axis_name="c", subcore_axis_name="s")
    return pl.kernel(
        _sc_body,
        out_type=jax.ShapeDtypeStruct((_L, 4, _NW, 8, 128), jnp.float32),
        mesh=mesh,
        compiler_params=pltpu.CompilerParams(use_tc_tiling_on_sc=False, needs_layout_passes=False),
        scratch_types=[
            pltpu.VMEM((_IPH,), jnp.int32),
            pltpu.VMEM((_IPH,), jnp.int32),
            pltpu.VMEM((_IPH, _E), jnp.bfloat16),
            pltpu.VMEM((_IPH, _E), jnp.bfloat16),
            pltpu.VMEM((_E, _BPW), jnp.float32),
            pltpu.VMEM((_E, _BPW), jnp.float32),
            pltpu.VMEM((_D, _E), jnp.bfloat16),
            pltpu.SemaphoreType.DMA,
            pltpu.SemaphoreType.DMA,
            pltpu.SemaphoreType.DMA,
            pltpu.SemaphoreType.DMA,
        ],
    )(syntax_lmaj, emb_table, pos_emb)


def kernel(syntax, emb_table, pos_emb):
    syntax_lmaj = syntax.transpose(1, 0, 2).reshape(_L * _B * _D)
    out5 = _syntax_embed(syntax_lmaj,
                         emb_table.astype(jnp.bfloat16),
                         pos_emb.astype(jnp.bfloat16))
    out = out5.transpose(2, 4, 0, 1, 3).reshape(_B, _L, _E)
    return out


# speedup vs baseline: 60.4439x; 1.0553x over previous
import functools

import jax
import jax.numpy as jnp
from jax import lax
from jax.experimental import pallas as pl
from jax.experimental.pallas import tpu as pltpu
from jax.experimental.pallas import tpu_sc as plsc

_B, _L, _D, _E = 4096, 50, 20, 32
_NW = 32
_BPW = _B // _NW
_HD = _D // 2
_GSZ = 128
_G = 4


def _sc_body(syntax_hbm, table_hbm, pos_hbm, out_hbm,
             idx_a, idx_b, slab_a, slab_b, acc_a, acc_b, pos_v,
             isem_a, isem_b, gsem_a, gsem_b, osem_a, osem_b):
    wid = lax.axis_index("s") * 2 + lax.axis_index("c")
    b0 = wid * _BPW
    pltpu.sync_copy(pos_hbm, pos_v)

    def idx_start(l, idx_v, isem):
        pltpu.async_copy(syntax_hbm.at[l, :, pl.ds(b0, _BPW)], idx_v, isem)

    def idx_wait(idx_v, isem):
        pltpu.make_async_copy(
            syntax_hbm.at[0, :, pl.ds(b0, _BPW)], idx_v, isem).wait()

    def fire(h, idx_v, slab, gsem):
        for j in range(_HD):
            pltpu.async_copy(
                table_hbm.at[idx_v.at[h * _HD + j]],
                slab.at[pl.ds(j * _GSZ, _GSZ)],
                gsem,
            )

    def drain(h, idx_v, slab, gsem):
        for j in range(_HD):
            pltpu.make_async_copy(
                table_hbm.at[idx_v.at[h * _HD + j]],
                slab.at[pl.ds(j * _GSZ, _GSZ)],
                gsem,
            ).wait()

    e_even = lax.iota(jnp.int32, 16) * 2
    e_odd = e_even + 1

    def compute(slab, acc, h):
        def group_body(g, _):
            racc = [jnp.zeros((32,), jnp.bfloat16) for _ in range(_G)]
            for dd in range(_HD):
                p = pos_v[h * _HD + dd, pl.ds(0, _E)]
                base = dd * _GSZ + g * _G
                for r in range(_G):
                    racc[r] += slab[base + r, pl.ds(0, _E)] * p
            for r in range(_G):
                j = jnp.full((16,), g * _G + r, jnp.int32)
                v_even, v_odd = plsc.unpack(racc[r],
                                            format=plsc.PackFormat.INTERLEAVED)
                if h == 0:
                    plsc.store_scatter(acc, [e_even, j], v_even)
                    plsc.store_scatter(acc, [e_odd, j], v_odd)
                else:
                    plsc.addupdate_scatter(acc, [e_even, j], v_even)
                    plsc.addupdate_scatter(acc, [e_odd, j], v_odd)
            return 0

        lax.fori_loop(0, _BPW // _G, group_body, 0, unroll=False)

    def out_wait(acc, osem):
        for k in range(4):
            pltpu.make_async_copy(
                acc.at[pl.ds(8 * k, 8)], out_hbm.at[0, k, wid], osem,
            ).wait()

    def out_send(l, acc, osem):
        for k in range(4):
            pltpu.async_copy(
                acc.at[pl.ds(8 * k, 8)], out_hbm.at[l, k, wid], osem,
            )

    idx_start(0, idx_a, isem_a)
    idx_wait(idx_a, isem_a)
    fire(0, idx_a, slab_a, gsem_a)
    idx_start(1, idx_b, isem_b)

    def pair_body(t, _):
        l0 = 2 * t
        l1 = l0 + 1
        last = t >= _L // 2 - 1

        fire(1, idx_a, slab_b, gsem_b)
        drain(0, idx_a, slab_a, gsem_a)

        @pl.when(t >= 1)
        def _():
            out_wait(acc_a, osem_a)

        compute(slab_a, acc_a, 0)

        idx_wait(idx_b, isem_b)
        fire(0, idx_b, slab_a, gsem_a)
        drain(1, idx_a, slab_b, gsem_b)
        compute(slab_b, acc_a, 1)
        out_send(l0, acc_a, osem_a)

        @pl.when(jnp.logical_not(last))
        def _():
            idx_start(l0 + 2, idx_a, isem_a)

        fire(1, idx_b, slab_b, gsem_b)
        drain(0, idx_b, slab_a, gsem_a)

        @pl.when(t >= 1)
        def _():
            out_wait(acc_b, osem_b)

        compute(slab_a, acc_b, 0)

        @pl.when(jnp.logical_not(last))
        def _():
            idx_wait(idx_a, isem_a)
            fire(0, idx_a, slab_a, gsem_a)

        drain(1, idx_b, slab_b, gsem_b)
        compute(slab_b, acc_b, 1)
        out_send(l1, acc_b, osem_b)

        @pl.when(jnp.logical_not(last))
        def _():
            idx_start(l1 + 2, idx_b, isem_b)

        return 0

    lax.fori_loop(0, _L // 2, pair_body, 0, unroll=False)
    out_wait(acc_a, osem_a)
    out_wait(acc_b, osem_b)


@jax.jit
def _syntax_embed(syntax_ldb, emb_table, pos_emb):
    mesh = plsc.VectorSubcoreMesh(core_axis_name="c", subcore_axis_name="s")
    return pl.kernel(
        _sc_body,
        out_type=jax.ShapeDtypeStruct((_L, 4, _NW, 8, 128), jnp.float32),
        mesh=mesh,
        compiler_params=pltpu.CompilerParams(use_tc_tiling_on_sc=False,
                                             needs_layout_passes=False),
        scratch_types=[
            pltpu.VMEM((_D, _BPW), jnp.int32),
            pltpu.VMEM((_D, _BPW), jnp.int32),
            pltpu.VMEM((_HD * _GSZ, _E), jnp.bfloat16),
            pltpu.VMEM((_HD * _GSZ, _E), jnp.bfloat16),
            pltpu.VMEM((_E, _BPW), jnp.float32),
            pltpu.VMEM((_E, _BPW), jnp.float32),
            pltpu.VMEM((_D, _E), jnp.bfloat16),
            pltpu.SemaphoreType.DMA,
            pltpu.SemaphoreType.DMA,
            pltpu.SemaphoreType.DMA,
            pltpu.SemaphoreType.DMA,
            pltpu.SemaphoreType.DMA,
            pltpu.SemaphoreType.DMA,
        ],
    )(syntax_ldb, emb_table, pos_emb)


def kernel(syntax, emb_table, pos_emb):
    syntax_ldb = syntax.transpose(1, 2, 0)
    out5 = _syntax_embed(syntax_ldb,
                         emb_table.astype(jnp.bfloat16),
                         pos_emb.astype(jnp.bfloat16))
    out = out5.transpose(2, 4, 0, 1, 3).reshape(_B, _L, _E)
    return out


# speedup vs baseline: 80.0429x; 1.3243x over previous
import functools

import jax
import jax.numpy as jnp
from jax import lax
from jax.experimental import pallas as pl
from jax.experimental.pallas import tpu as pltpu
from jax.experimental.pallas import tpu_sc as plsc

_B, _L, _D, _E = 4096, 50, 20, 32
_NW = 32
_BPW = _B // _NW
_GSZ = 128
_G = 4


def _sc_body(syntax_hbm, table_hbm, pos_hbm, out_hbm,
             idx_a, idx_b, slab_a, slab_b, acc_a, acc_b, pos_v,
             isem_a, isem_b, gsem_a, gsem_b, osem_a, osem_b):
    wid = lax.axis_index("s") * 2 + lax.axis_index("c")
    b0 = wid * _BPW
    pltpu.sync_copy(pos_hbm, pos_v)

    def idx_start(l, idx_v, isem):
        pltpu.async_copy(syntax_hbm.at[:, l, pl.ds(b0, _BPW)], idx_v, isem)

    def idx_wait(idx_v, isem):
        pltpu.make_async_copy(
            syntax_hbm.at[:, 0, pl.ds(b0, _BPW)], idx_v, isem).wait()

    def fire(idx_v, slab, gsem):
        for d in range(_D):
            pltpu.async_copy(
                table_hbm.at[idx_v.at[d]],
                slab.at[pl.ds(d * _GSZ, _GSZ)],
                gsem,
            )

    def drain(idx_v, slab, gsem):
        for d in range(_D):
            pltpu.make_async_copy(
                table_hbm.at[idx_v.at[d]],
                slab.at[pl.ds(d * _GSZ, _GSZ)],
                gsem,
            ).wait()

    e_even = lax.iota(jnp.int32, 16) * 2
    e_odd = e_even + 1

    def compute(slab, acc):
        def group_body(g, _):
            racc = [jnp.zeros((32,), jnp.bfloat16) for _ in range(_G)]
            for d in range(_D):
                p = pos_v[d, pl.ds(0, _E)]
                base = d * _GSZ + g * _G
                for r in range(_G):
                    racc[r] += slab[base + r, pl.ds(0, _E)] * p
            for r in range(_G):
                j = jnp.full((16,), g * _G + r, jnp.int32)
                v_even, v_odd = plsc.unpack(racc[r],
                                            format=plsc.PackFormat.INTERLEAVED)
                plsc.store_scatter(acc, [e_even, j], v_even)
                plsc.store_scatter(acc, [e_odd, j], v_odd)
            return 0

        lax.fori_loop(0, _BPW // _G, group_body, 0, unroll=False)

    def out_wait(acc, osem):
        for k in range(4):
            pltpu.make_async_copy(
                acc.at[pl.ds(8 * k, 8)], out_hbm.at[0, k, wid], osem,
            ).wait()

    def out_send(l, acc, osem):
        for k in range(4):
            pltpu.async_copy(
                acc.at[pl.ds(8 * k, 8)], out_hbm.at[l, k, wid], osem,
            )

    idx_start(0, idx_a, isem_a)
    idx_wait(idx_a, isem_a)
    fire(idx_a, slab_a, gsem_a)
    idx_start(1, idx_b, isem_b)

    def pair_body(t, _):
        l0 = 2 * t
        l1 = l0 + 1
        last = t >= _L // 2 - 1

        idx_wait(idx_b, isem_b)
        fire(idx_b, slab_b, gsem_b)
        drain(idx_a, slab_a, gsem_a)

        @pl.when(jnp.logical_not(last))
        def _():
            idx_start(l0 + 2, idx_a, isem_a)

        @pl.when(t >= 1)
        def _():
            out_wait(acc_a, osem_a)

        compute(slab_a, acc_a)
        out_send(l0, acc_a, osem_a)

        @pl.when(jnp.logical_not(last))
        def _():
            idx_wait(idx_a, isem_a)
            fire(idx_a, slab_a, gsem_a)

        drain(idx_b, slab_b, gsem_b)

        @pl.when(t >= 1)
        def _():
            out_wait(acc_b, osem_b)

        compute(slab_b, acc_b)
        out_send(l1, acc_b, osem_b)

        @pl.when(jnp.logical_not(last))
        def _():
            idx_start(l1 + 2, idx_b, isem_b)

        return 0

    lax.fori_loop(0, _L // 2, pair_body, 0, unroll=False)
    out_wait(acc_a, osem_a)
    out_wait(acc_b, osem_b)


@jax.jit
def _syntax_embed(syntax_dlb, emb_table, pos_emb):
    mesh = plsc.VectorSubcoreMesh(core_axis_name="c", subcore_axis_name="s")
    return pl.kernel(
        _sc_body,
        out_type=jax.ShapeDtypeStruct((_L, 4, _NW, 8, 128), jnp.float32),
        mesh=mesh,
        compiler_params=pltpu.CompilerParams(use_tc_tiling_on_sc=False,
                                             needs_layout_passes=False),
        scratch_types=[
            pltpu.VMEM((_D, _BPW), jnp.int32),
            pltpu.VMEM((_D, _BPW), jnp.int32),
            pltpu.VMEM((_D * _GSZ, _E), jnp.bfloat16),
            pltpu.VMEM((_D * _GSZ, _E), jnp.bfloat16),
            pltpu.VMEM((_E, _BPW), jnp.float32),
            pltpu.VMEM((_E, _BPW), jnp.float32),
            pltpu.VMEM((_D, _E), jnp.bfloat16),
            pltpu.SemaphoreType.DMA,
            pltpu.SemaphoreType.DMA,
            pltpu.SemaphoreType.DMA,
            pltpu.SemaphoreType.DMA,
            pltpu.SemaphoreType.DMA,
            pltpu.SemaphoreType.DMA,
        ],
    )(syntax_dlb, emb_table, pos_emb)


def kernel(syntax, emb_table, pos_emb):
    syntax_dlb = syntax.transpose(2, 1, 0)
    out5 = _syntax_embed(syntax_dlb,
                         emb_table.astype(jnp.bfloat16),
                         pos_emb.astype(jnp.bfloat16))
    out = out5.transpose(2, 4, 0, 1, 3).reshape(_B, _L, _E)
    return out


# speedup vs baseline: 83.2724x; 1.0403x over previous
import functools

import jax
import jax.numpy as jnp
from jax import lax
from jax.experimental import pallas as pl
from jax.experimental.pallas import tpu as pltpu
from jax.experimental.pallas import tpu_sc as plsc

_B, _L, _D, _E = 4096, 50, 20, 32
_NW = 32
_BPW = _B // _NW
_GSZ = 128
_G = 4


def _sc_body(syntax_hbm, table_hbm, pos_hbm, out_hbm,
             idx_a, idx_b, slab_a, slab_b, acc_a, acc_b, pos_v,
             isem_a, isem_b, gsem_a, gsem_b, osem_a, osem_b):
    wid = lax.axis_index("s") * 2 + lax.axis_index("c")
    b0 = wid * _BPW
    pltpu.sync_copy(pos_hbm, pos_v)

    def idx_start(l, idx_v, isem):
        pltpu.async_copy(syntax_hbm.at[:, l, pl.ds(b0, _BPW)], idx_v, isem)

    def idx_wait(idx_v, isem):
        pltpu.make_async_copy(
            syntax_hbm.at[:, 0, pl.ds(b0, _BPW)], idx_v, isem).wait()

    def fire(idx_v, slab, gsem):
        for d in range(_D):
            pltpu.async_copy(
                table_hbm.at[idx_v.at[d]],
                slab.at[pl.ds(d * _GSZ, _GSZ)],
                gsem,
            )

    def drain(idx_v, slab, gsem):
        for d in range(_D):
            pltpu.make_async_copy(
                table_hbm.at[idx_v.at[d]],
                slab.at[pl.ds(d * _GSZ, _GSZ)],
                gsem,
            ).wait()

    e_even = lax.iota(jnp.int32, 16) * 2
    e_odd = e_even + 1

    def compute(slab, acc):
        @plsc.parallel_loop(0, _BPW // _G, 1, unroll=2)
        def _(g):
            racc = [jnp.zeros((32,), jnp.bfloat16) for _ in range(_G)]
            for d in range(_D):
                p = pos_v[d, pl.ds(0, _E)]
                base = d * _GSZ + g * _G
                for r in range(_G):
                    racc[r] += slab[base + r, pl.ds(0, _E)] * p
            for r in range(_G):
                j = jnp.full((16,), g * _G + r, jnp.int32)
                v_even, v_odd = plsc.unpack(racc[r],
                                            format=plsc.PackFormat.INTERLEAVED)
                plsc.store_scatter(acc, [e_even, j], v_even)
                plsc.store_scatter(acc, [e_odd, j], v_odd)

    def out_wait(acc, osem):
        for k in range(4):
            pltpu.make_async_copy(
                acc.at[pl.ds(8 * k, 8)], out_hbm.at[0, k, wid], osem,
            ).wait()

    def out_send(l, acc, osem):
        for k in range(4):
            pltpu.async_copy(
                acc.at[pl.ds(8 * k, 8)], out_hbm.at[l, k, wid], osem,
            )

    idx_start(0, idx_a, isem_a)
    idx_wait(idx_a, isem_a)
    fire(idx_a, slab_a, gsem_a)
    idx_start(1, idx_b, isem_b)

    def pair_body(t, _):
        l0 = 2 * t
        l1 = l0 + 1
        last = t >= _L // 2 - 1

        idx_wait(idx_b, isem_b)
        fire(idx_b, slab_b, gsem_b)
        drain(idx_a, slab_a, gsem_a)

        @pl.when(jnp.logical_not(last))
        def _():
            idx_start(l0 + 2, idx_a, isem_a)

        @pl.when(t >= 1)
        def _():
            out_wait(acc_a, osem_a)

        compute(slab_a, acc_a)
        out_send(l0, acc_a, osem_a)

        @pl.when(jnp.logical_not(last))
        def _():
            idx_wait(idx_a, isem_a)
            fire(idx_a, slab_a, gsem_a)

        drain(idx_b, slab_b, gsem_b)

        @pl.when(t >= 1)
        def _():
            out_wait(acc_b, osem_b)

        compute(slab_b, acc_b)
        out_send(l1, acc_b, osem_b)

        @pl.when(jnp.logical_not(last))
        def _():
            idx_start(l1 + 2, idx_b, isem_b)

        return 0

    lax.fori_loop(0, _L // 2, pair_body, 0, unroll=False)
    out_wait(acc_a, osem_a)
    out_wait(acc_b, osem_b)


@jax.jit
def _syntax_embed(syntax_dlb, emb_table, pos_emb):
    mesh = plsc.VectorSubcoreMesh(core_axis_name="c", subcore_axis_name="s")
    return pl.kernel(
        _sc_body,
        out_type=jax.ShapeDtypeStruct((_L, 4, _NW, 8, 128), jnp.float32),
        mesh=mesh,
        compiler_params=pltpu.CompilerParams(use_tc_tiling_on_sc=False,
                                             needs_layout_passes=False),
        scratch_types=[
            pltpu.VMEM((_D, _BPW), jnp.int32),
            pltpu.VMEM((_D, _BPW), jnp.int32),
            pltpu.VMEM((_D * _GSZ, _E), jnp.bfloat16),
            pltpu.VMEM((_D * _GSZ, _E), jnp.bfloat16),
            pltpu.VMEM((_E, _BPW), jnp.float32),
            pltpu.VMEM((_E, _BPW), jnp.float32),
            pltpu.VMEM((_D, _E), jnp.bfloat16),
            pltpu.SemaphoreType.DMA,
            pltpu.SemaphoreType.DMA,
            pltpu.SemaphoreType.DMA,
            pltpu.SemaphoreType.DMA,
            pltpu.SemaphoreType.DMA,
            pltpu.SemaphoreType.DMA,
        ],
    )(syntax_dlb, emb_table, pos_emb)


def kernel(syntax, emb_table, pos_emb):
    syntax_dlb = syntax.transpose(2, 1, 0)
    table_bf = emb_table.astype(jnp.bfloat16)
    out5 = _syntax_embed(syntax_dlb, table_bf,
                         pos_emb.astype(jnp.bfloat16))
    out = out5.transpose(2, 4, 0, 1, 3).reshape(_B, _L, _E)
    return out
